# baseline (speedup 1.0000x reference)
import functools

import jax
import jax.numpy as jnp
from jax import lax
from jax.experimental import pallas as pl
from jax.experimental.pallas import tpu as pltpu
from jax.experimental.pallas import tpu_sc as plsc

_BATCH = 4096
_HIST = 200
_D = 64
_DP = 128
_NW = 32
_BPW = _BATCH // _NW
_NBUF = 4
_LAG = 3
_NT = _BPW // _NBUF

_mesh = plsc.VectorSubcoreMesh(core_axis_name="c", subcore_axis_name="s")


@functools.partial(
    pl.kernel,
    out_type=jax.ShapeDtypeStruct((_BATCH, _HIST, _D), jnp.float32),
    mesh=_mesh,
    scratch_types=[
        pltpu.VMEM((_NBUF, _HIST), jnp.int32),
        pltpu.VMEM((_NBUF, _HIST, _DP), jnp.float32),
        pltpu.SemaphoreType.DMA((_NBUF,)),
        pltpu.SemaphoreType.DMA((_NBUF,)),
        pltpu.SemaphoreType.DMA((_NBUF,)),
    ],
    compiler_params=pltpu.CompilerParams(use_tc_tiling_on_sc=False),
)
def _gather_kernel(ids_hbm, table_hbm, out_hbm, idx_v, rows_v,
                   isem, gsem, osem):
    wid = lax.axis_index("s") * 2 + lax.axis_index("c")
    base = wid * _BPW

    def fire_idx(b, buf):
        pltpu.async_copy(ids_hbm.at[pl.ds((base + b) * _HIST, _HIST)],
                         idx_v.at[buf], isem.at[buf])

    def wait_idx(buf):
        pltpu.make_async_copy(ids_hbm.at[pl.ds(0, _HIST)], idx_v.at[buf],
                              isem.at[buf]).wait()

    def fire_gather(buf):
        pltpu.async_copy(table_hbm.at[idx_v.at[buf]], rows_v.at[buf],
                         gsem.at[buf])

    def wait_gather(buf):
        pltpu.make_async_copy(table_hbm.at[idx_v.at[buf]], rows_v.at[buf],
                              gsem.at[buf]).wait()

    def fire_out(b, buf):
        pltpu.async_copy(rows_v.at[buf, :, pl.ds(0, _D)],
                         out_hbm.at[base + b], osem.at[buf])

    def wait_out(buf):
        pltpu.make_async_copy(rows_v.at[buf, :, pl.ds(0, _D)],
                              out_hbm.at[0], osem.at[buf]).wait()

    def drain(b, buf):
        wait_gather(buf)
        fire_out(b, buf)
        if isinstance(b, int):
            if b + _NBUF < _BPW:
                fire_idx(b + _NBUF, buf)
        else:
            @pl.when(b + _NBUF < _BPW)
            def _():
                fire_idx(b + _NBUF, buf)

    for buf in range(_NBUF):
        fire_idx(buf, buf)
    for j in range(_NBUF):
        wait_idx(j)
        fire_gather(j)
        if j >= _LAG:
            drain(j - _LAG, j - _LAG)

    def body(t, carry):
        for buf in range(_NBUF):
            b = t * _NBUF + buf
            wait_out(buf)
            wait_idx(buf)
            fire_gather(buf)
            drain(b - _LAG, (buf - _LAG) % _NBUF)
        return carry

    lax.fori_loop(1, _NT, body, 0)

    for j in range(_BPW - _LAG, _BPW):
        drain(j, j % _NBUF)
    for buf in range(_NBUF):
        wait_out(buf)


def kernel(item_ids, table):
    num_embeddings, d = table.shape
    ids = jnp.clip(item_ids, 0, num_embeddings - 1).reshape(-1)
    table_p = jnp.pad(table, ((0, 0), (0, _DP - d)))
    return _gather_kernel(ids, table_p)


# speedup vs baseline: 1.0462x; 1.0462x over previous
import functools

import jax
import jax.numpy as jnp
from jax import lax
from jax.experimental import pallas as pl
from jax.experimental.pallas import tpu as pltpu
from jax.experimental.pallas import tpu_sc as plsc

_B = 4096 * 200
_D = 64
_NW = 32
_BPW = _B // _NW
_C = 128
_NBUF = 8
_LAG = 6
_NCH = _BPW // _C
_NT = _NCH // _NBUF

_mesh = plsc.VectorSubcoreMesh(core_axis_name="c", subcore_axis_name="s")


@functools.partial(
    pl.kernel,
    out_type=jax.ShapeDtypeStruct((_B, _D), jnp.float32),
    mesh=_mesh,
    scratch_types=[
        pltpu.VMEM((_NBUF, _C), jnp.int32),
        pltpu.VMEM((_NBUF, _C, _D), jnp.float32),
        pltpu.SemaphoreType.DMA((_NBUF,)),
        pltpu.SemaphoreType.DMA((_NBUF,)),
        pltpu.SemaphoreType.DMA((_NBUF,)),
    ],
    compiler_params=pltpu.CompilerParams(use_tc_tiling_on_sc=False),
)
def _gather_kernel(ids_hbm, table_hbm, out_hbm, idx_v, rows_v,
                   isem, gsem, osem):
    wid = lax.axis_index("s") * 2 + lax.axis_index("c")
    base = wid * _BPW

    def fire_idx(g, b):
        pltpu.async_copy(ids_hbm.at[pl.ds(base + g * _C, _C)],
                         idx_v.at[b], isem.at[b])

    def wait_idx(b):
        pltpu.make_async_copy(ids_hbm.at[pl.ds(0, _C)], idx_v.at[b],
                              isem.at[b]).wait()

    def fire_gather(b):
        pltpu.async_copy(table_hbm.at[idx_v.at[b]], rows_v.at[b], gsem.at[b])

    def wait_gather(b):
        pltpu.make_async_copy(table_hbm.at[idx_v.at[b]], rows_v.at[b],
                              gsem.at[b]).wait()

    def fire_out(g, b):
        pltpu.async_copy(rows_v.at[b], out_hbm.at[pl.ds(base + g * _C, _C)],
                         osem.at[b])

    def wait_out(b):
        pltpu.make_async_copy(rows_v.at[b], out_hbm.at[pl.ds(0, _C)],
                              osem.at[b]).wait()

    def drain(g, b):
        wait_gather(b)
        fire_out(g, b)
        if isinstance(g, int):
            if g + _NBUF < _NCH:
                fire_idx(g + _NBUF, b)
        else:
            @pl.when(g + _NBUF < _NCH)
            def _():
                fire_idx(g + _NBUF, b)

    for b in range(_NBUF):
        fire_idx(b, b)
    for j in range(_NBUF):
        wait_idx(j)
        fire_gather(j)
        if j >= _LAG:
            drain(j - _LAG, j - _LAG)

    def body(t, carry):
        for b in range(_NBUF):
            g = t * _NBUF + b
            wait_out(b)
            wait_idx(b)
            fire_gather(b)
            drain(g - _LAG, (b - _LAG) % _NBUF)
        return carry

    lax.fori_loop(1, _NT, body, 0)

    for j in range(_NCH - _LAG, _NCH):
        drain(j, j % _NBUF)
    for b in range(_NBUF):
        wait_out(b)


def kernel(item_ids, table):
    num_embeddings = table.shape[0]
    ids = jnp.clip(item_ids.reshape(-1), 0, num_embeddings - 1)
    out = _gather_kernel(ids, table)
    return out.reshape(item_ids.shape + (table.shape[1],))


# speedup vs baseline: 1.2755x; 1.2192x over previous
import functools

import jax
import jax.numpy as jnp
from jax import lax
from jax.experimental import pallas as pl
from jax.experimental.pallas import tpu as pltpu
from jax.experimental.pallas import tpu_sc as plsc

_B = 4096 * 200
_D = 64
_DP = 128
_NW = 32
_BPW = _B // _NW
_C = 128
_NBUF = 5
_LAG = 4
_NCH = _BPW // _C
_NT = _NCH // _NBUF

_mesh = plsc.VectorSubcoreMesh(core_axis_name="c", subcore_axis_name="s")


@functools.partial(
    pl.kernel,
    out_type=jax.ShapeDtypeStruct((_B, _DP), jnp.float32),
    mesh=_mesh,
    scratch_types=[
        pltpu.VMEM((_NBUF, _C), jnp.int32),
        pltpu.VMEM((_NBUF, _C, _DP), jnp.float32),
        pltpu.SemaphoreType.DMA((_NBUF,)),
        pltpu.SemaphoreType.DMA((_NBUF,)),
        pltpu.SemaphoreType.DMA((_NBUF,)),
    ],
    compiler_params=pltpu.CompilerParams(use_tc_tiling_on_sc=True),
)
def _gather_kernel(ids_hbm, table_hbm, out_hbm, idx_v, rows_v,
                   isem, gsem, osem):
    wid = lax.axis_index("s") * 2 + lax.axis_index("c")
    base = wid * _BPW

    def fire_idx(g, b):
        pltpu.async_copy(ids_hbm.at[pl.ds(base + g * _C, _C)],
                         idx_v.at[b], isem.at[b])

    def wait_idx(b):
        pltpu.make_async_copy(ids_hbm.at[pl.ds(0, _C)], idx_v.at[b],
                              isem.at[b]).wait()

    def fire_gather(b):
        pltpu.async_copy(table_hbm.at[idx_v.at[b]], rows_v.at[b], gsem.at[b])

    def wait_gather(b):
        pltpu.make_async_copy(table_hbm.at[idx_v.at[b]], rows_v.at[b],
                              gsem.at[b]).wait()

    def fire_out(g, b):
        pltpu.async_copy(rows_v.at[b], out_hbm.at[pl.ds(base + g * _C, _C)],
                         osem.at[b])

    def wait_out(b):
        pltpu.make_async_copy(rows_v.at[b], out_hbm.at[pl.ds(0, _C)],
                              osem.at[b]).wait()

    def drain(g, b):
        wait_gather(b)
        fire_out(g, b)
        if isinstance(g, int):
            if g + _NBUF < _NCH:
                fire_idx(g + _NBUF, b)
        else:
            @pl.when(g + _NBUF < _NCH)
            def _():
                fire_idx(g + _NBUF, b)

    for b in range(_NBUF):
        fire_idx(b, b)
    for j in range(_NBUF):
        wait_idx(j)
        fire_gather(j)
        if j >= _LAG:
            drain(j - _LAG, j - _LAG)

    def body(t, carry):
        for b in range(_NBUF):
            g = t * _NBUF + b
            wait_out(b)
            wait_idx(b)
            fire_gather(b)
            drain(g - _LAG, (b - _LAG) % _NBUF)
        return carry

    lax.fori_loop(1, _NT, body, 0)

    for j in range(_NCH - _LAG, _NCH):
        drain(j, j % _NBUF)
    for b in range(_NBUF):
        wait_out(b)


def kernel(item_ids, table):
    num_embeddings, d = table.shape
    ids = jnp.clip(item_ids.reshape(-1), 0, num_embeddings - 1)
    table_p = jnp.pad(table, ((0, 0), (0, _DP - d)))
    out_p = _gather_kernel(ids, table_p)
    return out_p[:, :d].reshape(item_ids.shape + (d,))


# speedup vs baseline: 1.3017x; 1.0206x over previous
import functools

import jax
import jax.numpy as jnp
from jax import lax
from jax.experimental import pallas as pl
from jax.experimental.pallas import tpu as pltpu
from jax.experimental.pallas import tpu_sc as plsc

_B = 4096 * 200
_D = 64
_DP = 128
_NW = 32
_BPW = _B // _NW
_C = 128
_NBUF = 5
_LAG = 4
_NCH = _BPW // _C
_NT = _NCH // _NBUF

_mesh = plsc.VectorSubcoreMesh(core_axis_name="c", subcore_axis_name="s")


@functools.partial(
    pl.kernel,
    out_type=jax.ShapeDtypeStruct((_B, _DP), jnp.float32),
    mesh=_mesh,
    scratch_types=[
        pltpu.VMEM((_NBUF, _C), jnp.int32),
        pltpu.VMEM((_NBUF, _C, _DP), jnp.float32),
        pltpu.SemaphoreType.DMA((_NBUF,)),
        pltpu.SemaphoreType.DMA((_NBUF,)),
        pltpu.SemaphoreType.DMA((_NBUF,)),
    ],
    compiler_params=pltpu.CompilerParams(use_tc_tiling_on_sc=False),
)
def _gather_kernel(ids_hbm, table_hbm, out_hbm, idx_v, rows_v,
                   isem, gsem, osem):
    wid = lax.axis_index("s") * 2 + lax.axis_index("c")
    base = wid * _BPW

    def fire_idx(g, b):
        pltpu.async_copy(ids_hbm.at[pl.ds(base + g * _C, _C)],
                         idx_v.at[b], isem.at[b])

    def wait_idx(b):
        pltpu.make_async_copy(ids_hbm.at[pl.ds(0, _C)], idx_v.at[b],
                              isem.at[b]).wait()

    def fire_gather(b):
        pltpu.async_copy(table_hbm.at[idx_v.at[b]], rows_v.at[b], gsem.at[b])

    def wait_gather(b):
        pltpu.make_async_copy(table_hbm.at[idx_v.at[b]], rows_v.at[b],
                              gsem.at[b]).wait()

    def fire_out(g, b):
        pltpu.async_copy(rows_v.at[b, :, pl.ds(0, _D)],
                         out_hbm.at[pl.ds(base + g * _C, _C), pl.ds(0, _D)],
                         osem.at[b])

    def wait_out(b):
        pltpu.make_async_copy(rows_v.at[b, :, pl.ds(0, _D)],
                              out_hbm.at[pl.ds(0, _C), pl.ds(0, _D)],
                              osem.at[b]).wait()

    def drain(g, b):
        wait_gather(b)
        fire_out(g, b)
        if isinstance(g, int):
            if g + _NBUF < _NCH:
                fire_idx(g + _NBUF, b)
        else:
            @pl.when(g + _NBUF < _NCH)
            def _():
                fire_idx(g + _NBUF, b)

    for b in range(_NBUF):
        fire_idx(b, b)
    for j in range(_NBUF):
        wait_idx(j)
        fire_gather(j)
        if j >= _LAG:
            drain(j - _LAG, j - _LAG)

    def body(t, carry):
        for b in range(_NBUF):
            g = t * _NBUF + b
            wait_out(b)
            wait_idx(b)
            fire_gather(b)
            drain(g - _LAG, (b - _LAG) % _NBUF)
        return carry

    lax.fori_loop(1, _NT, body, 0)

    for j in range(_NCH - _LAG, _NCH):
        drain(j, j % _NBUF)
    for b in range(_NBUF):
        wait_out(b)


def kernel(item_ids, table):
    num_embeddings, d = table.shape
    ids = jnp.clip(item_ids.reshape(-1), 0, num_embeddings - 1)
    table_p = jnp.pad(table, ((0, 0), (0, _DP - d)))
    out_p = _gather_kernel(ids, table_p)
    return out_p[:, :d].reshape(item_ids.shape + (d,))
